# baseline (speedup 1.0000x reference)
import numpy as np
import jax
import jax.numpy as jnp
from jax.experimental import pallas as pl
from jax.experimental.pallas import tpu as pltpu

_EPS = 1e-5
_N0 = 1024.0 * 28 * 28
_N1 = 1024.0 * 14 * 14
_N2 = 1024.0 * 7 * 7
_T = 64
_ROWS = _T * 8

_S0 = np.zeros((168, 7 * 16, 9), np.float32)
for C in range(7):
    for ue in range(4):
        for vf in range(4):
            for i in range(3):
                for j in range(3):
                    d = ue + i - 1
                    cx = 4 * C + vf + j - 1
                    if not (0 <= cx < 28):
                        continue
                    if d == -1:
                        k = cx
                    elif d <= 3:
                        k = 28 + 28 * d + cx
                    else:
                        k = 140 + cx
                    _S0[k, C * 16 + ue * 4 + vf, 3 * i + j] = 1.0
_S1 = np.zeros((16, 4, 2, 2), np.float32)
for e in range(2):
    for f in range(2):
        for dr in range(2):
            for dc in range(2):
                _S1[(2 * e + dr) * 4 + (2 * f + dc), e * 2 + f, dr, dc] = 1.0
_BMASK = np.tile(np.array([1.0] * 7 + [0.0], np.float32).reshape(8, 1),
                 (_T, 1))
_APOOL = np.kron(np.eye(_T, dtype=np.float32),
                 np.array([[1.0 / 49.0] * 7 + [0.0]], np.float32))


def _fold_mats(groups, ch):
    F = np.kron(np.ones((groups, 1), np.float32), np.eye(ch, dtype=np.float32))
    return F, F.T.copy()


_F0, _G0 = _fold_mats(112, 32)
_F1, _G1 = _fold_mats(28, 64)
_F2, _G2 = _fold_mats(7, 128)


def _build_x(xs_ref):
    xb = xs_ref[...]
    A = xb[:, 0:8, 84:112]
    Bv = xb[:, 1:9, :]
    Cv = xb[:, 2:10, 0:28]
    return jnp.concatenate([A, Bv, Cv], axis=2).reshape(_ROWS, 168)


def _conv1(h0, w1a_ref, w1b_ref):
    parts = []
    for C in range(7):
        g = h0[:, C * 512:(C + 1) * 512]
        parts.append(jnp.dot(g[:, 0:256], w1a_ref[...],
                             preferred_element_type=jnp.float32))
        parts.append(jnp.dot(g[:, 256:512], w1b_ref[...],
                             preferred_element_type=jnp.float32))
    return jnp.concatenate(parts, axis=1)


def _conv2(h1, w2_ref):
    parts = []
    for C in range(7):
        g = h1[:, C * 256:(C + 1) * 256]
        parts.append(jnp.dot(g, w2_ref[...],
                             preferred_element_type=jnp.float32))
    return jnp.concatenate(parts, axis=1)


def _k_stats0(xs_ref, w_ref, m_ref, s_ref, q_ref):
    h = jnp.dot(_build_x(xs_ref), w_ref[...],
                preferred_element_type=jnp.float32)
    hm = h * m_ref[...]
    s_ref[0, 0, :] = jnp.sum(hm, axis=0)
    q_ref[0, 0, :] = jnp.sum(hm * h, axis=0)


def _bn_vecs(s_ref, q_ref, f_ref, g_ref, n, ga_ref, be_ref):
    s = jnp.sum(s_ref[...], axis=(0, 1)).reshape(1, -1)
    q = jnp.sum(q_ref[...], axis=(0, 1)).reshape(1, -1)
    sc = jnp.dot(s, f_ref[...], preferred_element_type=jnp.float32) / n
    qc = jnp.dot(q, f_ref[...], preferred_element_type=jnp.float32) / n
    var = qc - sc * sc
    scale = ga_ref[...] * jax.lax.rsqrt(var + _EPS)
    shift = be_ref[...] - sc * scale
    sct = jnp.dot(scale, g_ref[...], preferred_element_type=jnp.float32)
    sht = jnp.dot(shift, g_ref[...], preferred_element_type=jnp.float32)
    return sct, sht


def _k_stage1(xs_ref, w0_ref, s0_ref, q0_ref, f0_ref, g0_ref, ga_ref, be_ref,
              w1a_ref, w1b_ref, m_ref, h1_ref, s_ref, q_ref):
    sct, sht = _bn_vecs(s0_ref, q0_ref, f0_ref, g0_ref, _N0, ga_ref, be_ref)
    h0raw = jnp.dot(_build_x(xs_ref), w0_ref[...],
                    preferred_element_type=jnp.float32)
    h0 = jnp.maximum(h0raw * sct + sht, 0.0)
    h1 = _conv1(h0, w1a_ref, w1b_ref)
    h1_ref[...] = h1
    hm = h1 * m_ref[...]
    s_ref[0, 0, :] = jnp.sum(hm, axis=0)
    q_ref[0, 0, :] = jnp.sum(hm * h1, axis=0)


def _k_stage2(h1_ref, s1_ref, q1_ref, f1_ref, g1_ref, ga_ref, be_ref,
              w2_ref, m_ref, h2_ref, s_ref, q_ref):
    sct, sht = _bn_vecs(s1_ref, q1_ref, f1_ref, g1_ref, _N1, ga_ref, be_ref)
    h1 = jnp.maximum(h1_ref[...] * sct + sht, 0.0)
    h2 = _conv2(h1, w2_ref)
    h2_ref[...] = h2
    hm = h2 * m_ref[...]
    s_ref[0, 0, :] = jnp.sum(hm, axis=0)
    q_ref[0, 0, :] = jnp.sum(hm * h2, axis=0)


def _k_stage3(h2_ref, s2_ref, q2_ref, f2_ref, g2_ref, ga_ref, be_ref,
              ap_ref, wfc_ref, bfc_ref, o_ref):
    sct, sht = _bn_vecs(s2_ref, q2_ref, f2_ref, g2_ref, _N2, ga_ref, be_ref)
    h2 = jnp.maximum(h2_ref[...] * sct + sht, 0.0)
    acc = h2[:, 0:128]
    for C in range(1, 7):
        acc = acc + h2[:, C * 128:(C + 1) * 128]
    pooled = jnp.dot(ap_ref[...], acc, preferred_element_type=jnp.float32)
    o_ref[...] = jnp.dot(pooled, wfc_ref[...],
                         preferred_element_type=jnp.float32) + bfc_ref[0]


def kernel(x, W0, b0, g0, be0, W1, b1, g1, be1, W2, b2, g2, be2, Wfc, bfc):
    B = x.shape[0]
    nT = B // _T
    f32 = jnp.float32

    xs = jnp.pad(x.reshape(B, 7, 112), ((0, 0), (1, 8), (0, 0)))

    w0r = jnp.transpose(W0[:, 0], (1, 2, 0)).reshape(9, 32)
    W0g = jnp.einsum('kpn,nc->kpc', jnp.asarray(_S0), w0r).reshape(168, 3584)
    W1g = jnp.einsum('pqde,ocde->pcqo', jnp.asarray(_S1), W1).reshape(512, 256)
    W1a = W1g[0:256, 0:128]
    W1b = W1g[256:512, 128:256]
    W2g = jnp.transpose(W2, (2, 3, 1, 0)).reshape(256, 128)
    wfcT = jnp.transpose(Wfc)
    apool = jnp.asarray(_APOOL)
    mask = jnp.asarray(_BMASK)
    F0, G0 = jnp.asarray(_F0), jnp.asarray(_G0)
    F1, G1 = jnp.asarray(_F1), jnp.asarray(_G1)
    F2, G2 = jnp.asarray(_F2), jnp.asarray(_G2)

    cparams = pltpu.CompilerParams(dimension_semantics=("parallel",))

    s0, q0 = pl.pallas_call(
        _k_stats0,
        grid=(nT,),
        in_specs=[
            pl.BlockSpec((_T, 16, 112), lambda i: (i, 0, 0)),
            pl.BlockSpec((168, 3584), lambda i: (0, 0)),
            pl.BlockSpec((_ROWS, 1), lambda i: (0, 0)),
        ],
        out_specs=[
            pl.BlockSpec((1, 1, 3584), lambda i: (i, 0, 0)),
            pl.BlockSpec((1, 1, 3584), lambda i: (i, 0, 0)),
        ],
        out_shape=[
            jax.ShapeDtypeStruct((nT, 1, 3584), f32),
            jax.ShapeDtypeStruct((nT, 1, 3584), f32),
        ],
        compiler_params=cparams,
    )(xs, W0g, mask)

    h1p, s1, q1 = pl.pallas_call(
        _k_stage1,
        grid=(nT,),
        in_specs=[
            pl.BlockSpec((_T, 16, 112), lambda i: (i, 0, 0)),
            pl.BlockSpec((168, 3584), lambda i: (0, 0)),
            pl.BlockSpec((nT, 1, 3584), lambda i: (0, 0, 0)),
            pl.BlockSpec((nT, 1, 3584), lambda i: (0, 0, 0)),
            pl.BlockSpec((3584, 32), lambda i: (0, 0)),
            pl.BlockSpec((32, 3584), lambda i: (0, 0)),
            pl.BlockSpec((1, 32), lambda i: (0, 0)),
            pl.BlockSpec((1, 32), lambda i: (0, 0)),
            pl.BlockSpec((256, 128), lambda i: (0, 0)),
            pl.BlockSpec((256, 128), lambda i: (0, 0)),
            pl.BlockSpec((_ROWS, 1), lambda i: (0, 0)),
        ],
        out_specs=[
            pl.BlockSpec((_ROWS, 1792), lambda i: (i, 0)),
            pl.BlockSpec((1, 1, 1792), lambda i: (i, 0, 0)),
            pl.BlockSpec((1, 1, 1792), lambda i: (i, 0, 0)),
        ],
        out_shape=[
            jax.ShapeDtypeStruct((B * 8, 1792), f32),
            jax.ShapeDtypeStruct((nT, 1, 1792), f32),
            jax.ShapeDtypeStruct((nT, 1, 1792), f32),
        ],
        compiler_params=cparams,
    )(xs, W0g, s0, q0, F0, G0, g0.reshape(1, 32), be0.reshape(1, 32),
      W1a, W1b, mask)

    h2p, s2, q2 = pl.pallas_call(
        _k_stage2,
        grid=(nT,),
        in_specs=[
            pl.BlockSpec((_ROWS, 1792), lambda i: (i, 0)),
            pl.BlockSpec((nT, 1, 1792), lambda i: (0, 0, 0)),
            pl.BlockSpec((nT, 1, 1792), lambda i: (0, 0, 0)),
            pl.BlockSpec((1792, 64), lambda i: (0, 0)),
            pl.BlockSpec((64, 1792), lambda i: (0, 0)),
            pl.BlockSpec((1, 64), lambda i: (0, 0)),
            pl.BlockSpec((1, 64), lambda i: (0, 0)),
            pl.BlockSpec((256, 128), lambda i: (0, 0)),
            pl.BlockSpec((_ROWS, 1), lambda i: (0, 0)),
        ],
        out_specs=[
            pl.BlockSpec((_ROWS, 896), lambda i: (i, 0)),
            pl.BlockSpec((1, 1, 896), lambda i: (i, 0, 0)),
            pl.BlockSpec((1, 1, 896), lambda i: (i, 0, 0)),
        ],
        out_shape=[
            jax.ShapeDtypeStruct((B * 8, 896), f32),
            jax.ShapeDtypeStruct((nT, 1, 896), f32),
            jax.ShapeDtypeStruct((nT, 1, 896), f32),
        ],
        compiler_params=cparams,
    )(h1p, s1, q1, F1, G1, g1.reshape(1, 64), be1.reshape(1, 64), W2g, mask)

    out = pl.pallas_call(
        _k_stage3,
        grid=(nT,),
        in_specs=[
            pl.BlockSpec((_ROWS, 896), lambda i: (i, 0)),
            pl.BlockSpec((nT, 1, 896), lambda i: (0, 0, 0)),
            pl.BlockSpec((nT, 1, 896), lambda i: (0, 0, 0)),
            pl.BlockSpec((896, 128), lambda i: (0, 0)),
            pl.BlockSpec((128, 896), lambda i: (0, 0)),
            pl.BlockSpec((1, 128), lambda i: (0, 0)),
            pl.BlockSpec((1, 128), lambda i: (0, 0)),
            pl.BlockSpec((_T, _ROWS), lambda i: (0, 0)),
            pl.BlockSpec((128, 10), lambda i: (0, 0)),
            pl.BlockSpec((1, 10), lambda i: (0, 0)),
        ],
        out_specs=pl.BlockSpec((_T, 10), lambda i: (i, 0)),
        out_shape=jax.ShapeDtypeStruct((B, 10), f32),
        compiler_params=cparams,
    )(h2p, s2, q2, F2, G2, g2.reshape(1, 128), be2.reshape(1, 128),
      apool, wfcT, bfc.reshape(1, 10))
    return out


# speedup vs baseline: 1.0860x; 1.0860x over previous
import numpy as np
import jax
import jax.numpy as jnp
from jax.experimental import pallas as pl
from jax.experimental.pallas import tpu as pltpu

_EPS = 1e-5
_BF = jnp.bfloat16
_N0 = 1024.0 * 28 * 28
_N1 = 1024.0 * 14 * 14
_N2 = 1024.0 * 7 * 7
_T = 64
_ROWS = _T * 8

_S0 = np.zeros((168, 7 * 16, 9), np.float32)
for C in range(7):
    for ue in range(4):
        for vf in range(4):
            for i in range(3):
                for j in range(3):
                    d = ue + i - 1
                    cx = 4 * C + vf + j - 1
                    if not (0 <= cx < 28):
                        continue
                    if d == -1:
                        k = cx
                    elif d <= 3:
                        k = 28 + 28 * d + cx
                    else:
                        k = 140 + cx
                    _S0[k, C * 16 + ue * 4 + vf, 3 * i + j] = 1.0
_S1 = np.zeros((16, 4, 2, 2), np.float32)
for e in range(2):
    for f in range(2):
        for dr in range(2):
            for dc in range(2):
                _S1[(2 * e + dr) * 4 + (2 * f + dc), e * 2 + f, dr, dc] = 1.0
_BMASK = np.tile(np.array([1.0] * 7 + [0.0], np.float32).reshape(8, 1),
                 (_T, 1))
_APOOL = np.kron(np.eye(_T, dtype=np.float32),
                 np.array([[1.0 / 49.0] * 7 + [0.0]], np.float32))


def _fold_mats(groups, ch):
    F = np.kron(np.ones((groups, 1), np.float32), np.eye(ch, dtype=np.float32))
    return F, F.T.copy()


_F0, _G0 = _fold_mats(112, 32)
_F1, _G1 = _fold_mats(28, 64)
_F2, _G2 = _fold_mats(7, 128)


def _build_x(xs_ref):
    xb = xs_ref[...]
    A = xb[:, 0:8, 84:112]
    Bv = xb[:, 1:9, :]
    Cv = xb[:, 2:10, 0:28]
    return jnp.concatenate([A, Bv, Cv], axis=2).reshape(_ROWS, 168)


def _conv1(h0, w1a_ref, w1b_ref):
    parts = []
    for C in range(7):
        g = h0[:, C * 512:(C + 1) * 512]
        parts.append(jnp.dot(g[:, 0:256], w1a_ref[...],
                             preferred_element_type=jnp.float32))
        parts.append(jnp.dot(g[:, 256:512], w1b_ref[...],
                             preferred_element_type=jnp.float32))
    return jnp.concatenate(parts, axis=1)


def _conv2(h1, w2_ref):
    parts = []
    for C in range(7):
        g = h1[:, C * 256:(C + 1) * 256]
        parts.append(jnp.dot(g, w2_ref[...],
                             preferred_element_type=jnp.float32))
    return jnp.concatenate(parts, axis=1)


def _k_stats0(xs_ref, w_ref, m_ref, s_ref, q_ref):
    h = jnp.dot(_build_x(xs_ref).astype(_BF), w_ref[...],
                preferred_element_type=jnp.float32)
    hm = h * m_ref[...]
    s_ref[0, 0, :] = jnp.sum(hm, axis=0)
    q_ref[0, 0, :] = jnp.sum(hm * h, axis=0)


def _bn_vecs(s_ref, q_ref, f_ref, g_ref, n, ga_ref, be_ref):
    s = jnp.sum(s_ref[...], axis=(0, 1)).reshape(1, -1)
    q = jnp.sum(q_ref[...], axis=(0, 1)).reshape(1, -1)
    sc = jnp.dot(s, f_ref[...], preferred_element_type=jnp.float32) / n
    qc = jnp.dot(q, f_ref[...], preferred_element_type=jnp.float32) / n
    var = qc - sc * sc
    scale = ga_ref[...] * jax.lax.rsqrt(var + _EPS)
    shift = be_ref[...] - sc * scale
    sct = jnp.dot(scale, g_ref[...], preferred_element_type=jnp.float32)
    sht = jnp.dot(shift, g_ref[...], preferred_element_type=jnp.float32)
    return sct, sht


def _k_stage1(xs_ref, w0_ref, s0_ref, q0_ref, f0_ref, g0_ref, ga_ref, be_ref,
              w1a_ref, w1b_ref, m_ref, h1_ref, s_ref, q_ref):
    sct, sht = _bn_vecs(s0_ref, q0_ref, f0_ref, g0_ref, _N0, ga_ref, be_ref)
    h0raw = jnp.dot(_build_x(xs_ref).astype(_BF), w0_ref[...],
                    preferred_element_type=jnp.float32)
    h0 = jnp.maximum(h0raw * sct + sht, 0.0).astype(_BF)
    h1 = _conv1(h0, w1a_ref, w1b_ref)
    h1b = h1.astype(_BF)
    h1_ref[...] = h1b
    h1f = h1b.astype(jnp.float32)
    hm = h1f * m_ref[...]
    s_ref[0, 0, :] = jnp.sum(hm, axis=0)
    q_ref[0, 0, :] = jnp.sum(hm * h1f, axis=0)


def _k_stage2(h1_ref, s1_ref, q1_ref, f1_ref, g1_ref, ga_ref, be_ref,
              w2_ref, m_ref, h2_ref, s_ref, q_ref):
    sct, sht = _bn_vecs(s1_ref, q1_ref, f1_ref, g1_ref, _N1, ga_ref, be_ref)
    h1 = jnp.maximum(h1_ref[...].astype(jnp.float32) * sct + sht,
                     0.0).astype(_BF)
    h2 = _conv2(h1, w2_ref)
    h2b = h2.astype(_BF)
    h2_ref[...] = h2b
    h2f = h2b.astype(jnp.float32)
    hm = h2f * m_ref[...]
    s_ref[0, 0, :] = jnp.sum(hm, axis=0)
    q_ref[0, 0, :] = jnp.sum(hm * h2f, axis=0)


def _k_stage3(h2_ref, s2_ref, q2_ref, f2_ref, g2_ref, ga_ref, be_ref,
              ap_ref, wfc_ref, bfc_ref, o_ref):
    sct, sht = _bn_vecs(s2_ref, q2_ref, f2_ref, g2_ref, _N2, ga_ref, be_ref)
    h2 = jnp.maximum(h2_ref[...].astype(jnp.float32) * sct + sht, 0.0)
    acc = h2[:, 0:128]
    for C in range(1, 7):
        acc = acc + h2[:, C * 128:(C + 1) * 128]
    pooled = jnp.dot(ap_ref[...], acc, preferred_element_type=jnp.float32)
    o_ref[...] = jnp.dot(pooled, wfc_ref[...],
                         preferred_element_type=jnp.float32) + bfc_ref[0]


def kernel(x, W0, b0, g0, be0, W1, b1, g1, be1, W2, b2, g2, be2, Wfc, bfc):
    B = x.shape[0]
    nT = B // _T
    f32 = jnp.float32

    xs = jnp.pad(x.reshape(B, 7, 112), ((0, 0), (1, 8), (0, 0)))

    w0r = jnp.transpose(W0[:, 0], (1, 2, 0)).reshape(9, 32)
    W0g = jnp.einsum('kpn,nc->kpc', jnp.asarray(_S0),
                     w0r).reshape(168, 3584).astype(_BF)
    W1g = jnp.einsum('pqde,ocde->pcqo', jnp.asarray(_S1),
                     W1).reshape(512, 256).astype(_BF)
    W1a = W1g[0:256, 0:128]
    W1b = W1g[256:512, 128:256]
    W2g = jnp.transpose(W2, (2, 3, 1, 0)).reshape(256, 128).astype(_BF)
    wfcT = jnp.transpose(Wfc)
    apool = jnp.asarray(_APOOL)
    mask = jnp.asarray(_BMASK)
    F0, G0 = jnp.asarray(_F0), jnp.asarray(_G0)
    F1, G1 = jnp.asarray(_F1), jnp.asarray(_G1)
    F2, G2 = jnp.asarray(_F2), jnp.asarray(_G2)

    cparams = pltpu.CompilerParams(dimension_semantics=("parallel",))

    s0, q0 = pl.pallas_call(
        _k_stats0,
        grid=(nT,),
        in_specs=[
            pl.BlockSpec((_T, 16, 112), lambda i: (i, 0, 0)),
            pl.BlockSpec((168, 3584), lambda i: (0, 0)),
            pl.BlockSpec((_ROWS, 1), lambda i: (0, 0)),
        ],
        out_specs=[
            pl.BlockSpec((1, 1, 3584), lambda i: (i, 0, 0)),
            pl.BlockSpec((1, 1, 3584), lambda i: (i, 0, 0)),
        ],
        out_shape=[
            jax.ShapeDtypeStruct((nT, 1, 3584), f32),
            jax.ShapeDtypeStruct((nT, 1, 3584), f32),
        ],
        compiler_params=cparams,
    )(xs, W0g, mask)

    h1p, s1, q1 = pl.pallas_call(
        _k_stage1,
        grid=(nT,),
        in_specs=[
            pl.BlockSpec((_T, 16, 112), lambda i: (i, 0, 0)),
            pl.BlockSpec((168, 3584), lambda i: (0, 0)),
            pl.BlockSpec((nT, 1, 3584), lambda i: (0, 0, 0)),
            pl.BlockSpec((nT, 1, 3584), lambda i: (0, 0, 0)),
            pl.BlockSpec((3584, 32), lambda i: (0, 0)),
            pl.BlockSpec((32, 3584), lambda i: (0, 0)),
            pl.BlockSpec((1, 32), lambda i: (0, 0)),
            pl.BlockSpec((1, 32), lambda i: (0, 0)),
            pl.BlockSpec((256, 128), lambda i: (0, 0)),
            pl.BlockSpec((256, 128), lambda i: (0, 0)),
            pl.BlockSpec((_ROWS, 1), lambda i: (0, 0)),
        ],
        out_specs=[
            pl.BlockSpec((_ROWS, 1792), lambda i: (i, 0)),
            pl.BlockSpec((1, 1, 1792), lambda i: (i, 0, 0)),
            pl.BlockSpec((1, 1, 1792), lambda i: (i, 0, 0)),
        ],
        out_shape=[
            jax.ShapeDtypeStruct((B * 8, 1792), _BF),
            jax.ShapeDtypeStruct((nT, 1, 1792), f32),
            jax.ShapeDtypeStruct((nT, 1, 1792), f32),
        ],
        compiler_params=cparams,
    )(xs, W0g, s0, q0, F0, G0, g0.reshape(1, 32), be0.reshape(1, 32),
      W1a, W1b, mask)

    h2p, s2, q2 = pl.pallas_call(
        _k_stage2,
        grid=(nT,),
        in_specs=[
            pl.BlockSpec((_ROWS, 1792), lambda i: (i, 0)),
            pl.BlockSpec((nT, 1, 1792), lambda i: (0, 0, 0)),
            pl.BlockSpec((nT, 1, 1792), lambda i: (0, 0, 0)),
            pl.BlockSpec((1792, 64), lambda i: (0, 0)),
            pl.BlockSpec((64, 1792), lambda i: (0, 0)),
            pl.BlockSpec((1, 64), lambda i: (0, 0)),
            pl.BlockSpec((1, 64), lambda i: (0, 0)),
            pl.BlockSpec((256, 128), lambda i: (0, 0)),
            pl.BlockSpec((_ROWS, 1), lambda i: (0, 0)),
        ],
        out_specs=[
            pl.BlockSpec((_ROWS, 896), lambda i: (i, 0)),
            pl.BlockSpec((1, 1, 896), lambda i: (i, 0, 0)),
            pl.BlockSpec((1, 1, 896), lambda i: (i, 0, 0)),
        ],
        out_shape=[
            jax.ShapeDtypeStruct((B * 8, 896), _BF),
            jax.ShapeDtypeStruct((nT, 1, 896), f32),
            jax.ShapeDtypeStruct((nT, 1, 896), f32),
        ],
        compiler_params=cparams,
    )(h1p, s1, q1, F1, G1, g1.reshape(1, 64), be1.reshape(1, 64), W2g, mask)

    out = pl.pallas_call(
        _k_stage3,
        grid=(nT,),
        in_specs=[
            pl.BlockSpec((_ROWS, 896), lambda i: (i, 0)),
            pl.BlockSpec((nT, 1, 896), lambda i: (0, 0, 0)),
            pl.BlockSpec((nT, 1, 896), lambda i: (0, 0, 0)),
            pl.BlockSpec((896, 128), lambda i: (0, 0)),
            pl.BlockSpec((128, 896), lambda i: (0, 0)),
            pl.BlockSpec((1, 128), lambda i: (0, 0)),
            pl.BlockSpec((1, 128), lambda i: (0, 0)),
            pl.BlockSpec((_T, _ROWS), lambda i: (0, 0)),
            pl.BlockSpec((128, 10), lambda i: (0, 0)),
            pl.BlockSpec((1, 10), lambda i: (0, 0)),
        ],
        out_specs=pl.BlockSpec((_T, 10), lambda i: (i, 0)),
        out_shape=jax.ShapeDtypeStruct((B, 10), f32),
        compiler_params=cparams,
    )(h2p, s2, q2, F2, G2, g2.reshape(1, 128), be2.reshape(1, 128),
      apool, wfcT, bfc.reshape(1, 10))
    return out


# speedup vs baseline: 1.2384x; 1.1403x over previous
import numpy as np
import jax
import jax.numpy as jnp
from jax.experimental import pallas as pl
from jax.experimental.pallas import tpu as pltpu

_EPS = 1e-5
_BF = jnp.bfloat16
_N0 = 1024.0 * 28 * 28
_N1 = 1024.0 * 14 * 14
_N2 = 1024.0 * 7 * 7
_T = 128
_ROWS = _T * 8

_S0 = np.zeros((168, 7 * 16, 9), np.float32)
for C in range(7):
    for ue in range(4):
        for vf in range(4):
            for i in range(3):
                for j in range(3):
                    d = ue + i - 1
                    cx = 4 * C + vf + j - 1
                    if not (0 <= cx < 28):
                        continue
                    if d == -1:
                        k = cx
                    elif d <= 3:
                        k = 28 + 28 * d + cx
                    else:
                        k = 140 + cx
                    _S0[k, C * 16 + ue * 4 + vf, 3 * i + j] = 1.0
_S1 = np.zeros((16, 4, 2, 2), np.float32)
for e in range(2):
    for f in range(2):
        for dr in range(2):
            for dc in range(2):
                _S1[(2 * e + dr) * 4 + (2 * f + dc), e * 2 + f, dr, dc] = 1.0
_BMASK = np.tile(np.array([1.0] * 7 + [0.0], np.float32).reshape(8, 1),
                 (_T, 1))
_APOOL = np.kron(np.eye(_T, dtype=np.float32),
                 np.array([[1.0 / 49.0] * 7 + [0.0]], np.float32))


def _fold_mats(groups, ch):
    F = np.kron(np.ones((groups, 1), np.float32), np.eye(ch, dtype=np.float32))
    return F, F.T.copy()


_F0, _G0 = _fold_mats(112, 32)
_F1, _G1 = _fold_mats(28, 64)
_F2, _G2 = _fold_mats(7, 128)


def _build_x(xs_ref):
    xb = xs_ref[...]
    A = xb[:, 0:8, 84:112]
    Bv = xb[:, 1:9, :]
    Cv = xb[:, 2:10, 0:28]
    return jnp.concatenate([A, Bv, Cv], axis=2).reshape(_ROWS, 168)


def _conv1(h0, w1a_ref, w1b_ref):
    parts = []
    for C in range(7):
        g = h0[:, C * 512:(C + 1) * 512]
        parts.append(jnp.dot(g[:, 0:256], w1a_ref[...],
                             preferred_element_type=jnp.float32))
        parts.append(jnp.dot(g[:, 256:512], w1b_ref[...],
                             preferred_element_type=jnp.float32))
    return jnp.concatenate(parts, axis=1)


def _conv2(h1, w2_ref):
    parts = []
    for C in range(7):
        g = h1[:, C * 256:(C + 1) * 256]
        parts.append(jnp.dot(g, w2_ref[...],
                             preferred_element_type=jnp.float32))
    return jnp.concatenate(parts, axis=1)


def _k_stats0(xs_ref, w_ref, m_ref, s_ref, q_ref):
    h = jnp.dot(_build_x(xs_ref).astype(_BF), w_ref[...],
                preferred_element_type=jnp.float32)
    hm = h * m_ref[...]
    s_ref[0, 0, :] = jnp.sum(hm, axis=0)
    q_ref[0, 0, :] = jnp.sum(hm * h, axis=0)


def _bn_vecs(s_ref, q_ref, f_ref, g_ref, n, ga_ref, be_ref):
    s = jnp.sum(s_ref[...], axis=(0, 1)).reshape(1, -1)
    q = jnp.sum(q_ref[...], axis=(0, 1)).reshape(1, -1)
    sc = jnp.dot(s, f_ref[...], preferred_element_type=jnp.float32) / n
    qc = jnp.dot(q, f_ref[...], preferred_element_type=jnp.float32) / n
    var = qc - sc * sc
    scale = ga_ref[...] * jax.lax.rsqrt(var + _EPS)
    shift = be_ref[...] - sc * scale
    sct = jnp.dot(scale, g_ref[...], preferred_element_type=jnp.float32)
    sht = jnp.dot(shift, g_ref[...], preferred_element_type=jnp.float32)
    return sct, sht


def _k_stage1(xs_ref, w0_ref, s0_ref, q0_ref, f0_ref, g0_ref, ga_ref, be_ref,
              w1a_ref, w1b_ref, m_ref, h1_ref, s_ref, q_ref):
    sct, sht = _bn_vecs(s0_ref, q0_ref, f0_ref, g0_ref, _N0, ga_ref, be_ref)
    h0raw = jnp.dot(_build_x(xs_ref).astype(_BF), w0_ref[...],
                    preferred_element_type=jnp.float32)
    h0 = jnp.maximum(h0raw * sct + sht, 0.0).astype(_BF)
    h1 = _conv1(h0, w1a_ref, w1b_ref)
    h1b = h1.astype(_BF)
    h1_ref[...] = h1b
    h1f = h1b.astype(jnp.float32)
    hm = h1f * m_ref[...]
    s_ref[0, 0, :] = jnp.sum(hm, axis=0)
    q_ref[0, 0, :] = jnp.sum(hm * h1f, axis=0)


def _k_stage2(h1_ref, s1_ref, q1_ref, f1_ref, g1_ref, ga_ref, be_ref,
              w2_ref, m_ref, h2_ref, s_ref, q_ref):
    sct, sht = _bn_vecs(s1_ref, q1_ref, f1_ref, g1_ref, _N1, ga_ref, be_ref)
    h1 = jnp.maximum(h1_ref[...].astype(jnp.float32) * sct + sht,
                     0.0).astype(_BF)
    h2 = _conv2(h1, w2_ref)
    h2b = h2.astype(_BF)
    h2_ref[...] = h2b
    h2f = h2b.astype(jnp.float32)
    hm = h2f * m_ref[...]
    s_ref[0, 0, :] = jnp.sum(hm, axis=0)
    q_ref[0, 0, :] = jnp.sum(hm * h2f, axis=0)


def _k_stage3(h2_ref, s2_ref, q2_ref, f2_ref, g2_ref, ga_ref, be_ref,
              ap_ref, wfc_ref, bfc_ref, o_ref):
    sct, sht = _bn_vecs(s2_ref, q2_ref, f2_ref, g2_ref, _N2, ga_ref, be_ref)
    h2 = jnp.maximum(h2_ref[...].astype(jnp.float32) * sct + sht, 0.0)
    acc = h2[:, 0:128]
    for C in range(1, 7):
        acc = acc + h2[:, C * 128:(C + 1) * 128]
    pooled = jnp.dot(ap_ref[...], acc, preferred_element_type=jnp.float32)
    o_ref[...] = jnp.dot(pooled, wfc_ref[...],
                         preferred_element_type=jnp.float32) + bfc_ref[0]


def kernel(x, W0, b0, g0, be0, W1, b1, g1, be1, W2, b2, g2, be2, Wfc, bfc):
    B = x.shape[0]
    nT = B // _T
    f32 = jnp.float32

    xs = jnp.pad(x.reshape(B, 7, 112), ((0, 0), (1, 8), (0, 0)))

    w0r = jnp.transpose(W0[:, 0], (1, 2, 0)).reshape(9, 32)
    W0g = jnp.einsum('kpn,nc->kpc', jnp.asarray(_S0),
                     w0r).reshape(168, 3584).astype(_BF)
    W1g = jnp.einsum('pqde,ocde->pcqo', jnp.asarray(_S1),
                     W1).reshape(512, 256).astype(_BF)
    W1a = W1g[0:256, 0:128]
    W1b = W1g[256:512, 128:256]
    W2g = jnp.transpose(W2, (2, 3, 1, 0)).reshape(256, 128).astype(_BF)
    wfcT = jnp.transpose(Wfc)
    apool = jnp.asarray(_APOOL)
    mask = jnp.asarray(_BMASK)
    F0, G0 = jnp.asarray(_F0), jnp.asarray(_G0)
    F1, G1 = jnp.asarray(_F1), jnp.asarray(_G1)
    F2, G2 = jnp.asarray(_F2), jnp.asarray(_G2)

    cparams = pltpu.CompilerParams(dimension_semantics=("parallel",))

    s0, q0 = pl.pallas_call(
        _k_stats0,
        grid=(nT,),
        in_specs=[
            pl.BlockSpec((_T, 16, 112), lambda i: (i, 0, 0)),
            pl.BlockSpec((168, 3584), lambda i: (0, 0)),
            pl.BlockSpec((_ROWS, 1), lambda i: (0, 0)),
        ],
        out_specs=[
            pl.BlockSpec((1, 1, 3584), lambda i: (i, 0, 0)),
            pl.BlockSpec((1, 1, 3584), lambda i: (i, 0, 0)),
        ],
        out_shape=[
            jax.ShapeDtypeStruct((nT, 1, 3584), f32),
            jax.ShapeDtypeStruct((nT, 1, 3584), f32),
        ],
        compiler_params=cparams,
    )(xs, W0g, mask)

    h1p, s1, q1 = pl.pallas_call(
        _k_stage1,
        grid=(nT,),
        in_specs=[
            pl.BlockSpec((_T, 16, 112), lambda i: (i, 0, 0)),
            pl.BlockSpec((168, 3584), lambda i: (0, 0)),
            pl.BlockSpec((nT, 1, 3584), lambda i: (0, 0, 0)),
            pl.BlockSpec((nT, 1, 3584), lambda i: (0, 0, 0)),
            pl.BlockSpec((3584, 32), lambda i: (0, 0)),
            pl.BlockSpec((32, 3584), lambda i: (0, 0)),
            pl.BlockSpec((1, 32), lambda i: (0, 0)),
            pl.BlockSpec((1, 32), lambda i: (0, 0)),
            pl.BlockSpec((256, 128), lambda i: (0, 0)),
            pl.BlockSpec((256, 128), lambda i: (0, 0)),
            pl.BlockSpec((_ROWS, 1), lambda i: (0, 0)),
        ],
        out_specs=[
            pl.BlockSpec((_ROWS, 1792), lambda i: (i, 0)),
            pl.BlockSpec((1, 1, 1792), lambda i: (i, 0, 0)),
            pl.BlockSpec((1, 1, 1792), lambda i: (i, 0, 0)),
        ],
        out_shape=[
            jax.ShapeDtypeStruct((B * 8, 1792), _BF),
            jax.ShapeDtypeStruct((nT, 1, 1792), f32),
            jax.ShapeDtypeStruct((nT, 1, 1792), f32),
        ],
        compiler_params=cparams,
    )(xs, W0g, s0, q0, F0, G0, g0.reshape(1, 32), be0.reshape(1, 32),
      W1a, W1b, mask)

    h2p, s2, q2 = pl.pallas_call(
        _k_stage2,
        grid=(nT,),
        in_specs=[
            pl.BlockSpec((_ROWS, 1792), lambda i: (i, 0)),
            pl.BlockSpec((nT, 1, 1792), lambda i: (0, 0, 0)),
            pl.BlockSpec((nT, 1, 1792), lambda i: (0, 0, 0)),
            pl.BlockSpec((1792, 64), lambda i: (0, 0)),
            pl.BlockSpec((64, 1792), lambda i: (0, 0)),
            pl.BlockSpec((1, 64), lambda i: (0, 0)),
            pl.BlockSpec((1, 64), lambda i: (0, 0)),
            pl.BlockSpec((256, 128), lambda i: (0, 0)),
            pl.BlockSpec((_ROWS, 1), lambda i: (0, 0)),
        ],
        out_specs=[
            pl.BlockSpec((_ROWS, 896), lambda i: (i, 0)),
            pl.BlockSpec((1, 1, 896), lambda i: (i, 0, 0)),
            pl.BlockSpec((1, 1, 896), lambda i: (i, 0, 0)),
        ],
        out_shape=[
            jax.ShapeDtypeStruct((B * 8, 896), _BF),
            jax.ShapeDtypeStruct((nT, 1, 896), f32),
            jax.ShapeDtypeStruct((nT, 1, 896), f32),
        ],
        compiler_params=cparams,
    )(h1p, s1, q1, F1, G1, g1.reshape(1, 64), be1.reshape(1, 64), W2g, mask)

    out = pl.pallas_call(
        _k_stage3,
        grid=(nT,),
        in_specs=[
            pl.BlockSpec((_ROWS, 896), lambda i: (i, 0)),
            pl.BlockSpec((nT, 1, 896), lambda i: (0, 0, 0)),
            pl.BlockSpec((nT, 1, 896), lambda i: (0, 0, 0)),
            pl.BlockSpec((896, 128), lambda i: (0, 0)),
            pl.BlockSpec((128, 896), lambda i: (0, 0)),
            pl.BlockSpec((1, 128), lambda i: (0, 0)),
            pl.BlockSpec((1, 128), lambda i: (0, 0)),
            pl.BlockSpec((_T, _ROWS), lambda i: (0, 0)),
            pl.BlockSpec((128, 10), lambda i: (0, 0)),
            pl.BlockSpec((1, 10), lambda i: (0, 0)),
        ],
        out_specs=pl.BlockSpec((_T, 10), lambda i: (i, 0)),
        out_shape=jax.ShapeDtypeStruct((B, 10), f32),
        compiler_params=cparams,
    )(h2p, s2, q2, F2, G2, g2.reshape(1, 128), be2.reshape(1, 128),
      apool, wfcT, bfc.reshape(1, 10))
    return out


# speedup vs baseline: 1.2828x; 1.0359x over previous
import numpy as np
import jax
import jax.numpy as jnp
from jax.experimental import pallas as pl
from jax.experimental.pallas import tpu as pltpu

_EPS = 1e-5
_BF = jnp.bfloat16
_N0 = 1024.0 * 28 * 28
_N1 = 1024.0 * 14 * 14
_N2 = 1024.0 * 7 * 7
_T = 256
_ROWS = _T * 8

_S0 = np.zeros((168, 7 * 16, 9), np.float32)
for C in range(7):
    for ue in range(4):
        for vf in range(4):
            for i in range(3):
                for j in range(3):
                    d = ue + i - 1
                    cx = 4 * C + vf + j - 1
                    if not (0 <= cx < 28):
                        continue
                    if d == -1:
                        k = cx
                    elif d <= 3:
                        k = 28 + 28 * d + cx
                    else:
                        k = 140 + cx
                    _S0[k, C * 16 + ue * 4 + vf, 3 * i + j] = 1.0
_S1 = np.zeros((16, 4, 2, 2), np.float32)
for e in range(2):
    for f in range(2):
        for dr in range(2):
            for dc in range(2):
                _S1[(2 * e + dr) * 4 + (2 * f + dc), e * 2 + f, dr, dc] = 1.0
_BMASK = np.tile(np.array([1.0] * 7 + [0.0], np.float32).reshape(8, 1),
                 (_T, 1))
_APOOL = np.kron(np.eye(_T, dtype=np.float32),
                 np.array([[1.0 / 49.0] * 7 + [0.0]], np.float32))


def _fold_mats(groups, ch):
    F = np.kron(np.ones((groups, 1), np.float32), np.eye(ch, dtype=np.float32))
    return F, F.T.copy()


_F0, _G0 = _fold_mats(112, 32)
_F1, _G1 = _fold_mats(28, 64)
_F2, _G2 = _fold_mats(7, 128)


def _build_x(xs_ref):
    xb = xs_ref[...]
    A = xb[:, 0:8, 84:112]
    Bv = xb[:, 1:9, :]
    Cv = xb[:, 2:10, 0:28]
    return jnp.concatenate([A, Bv, Cv], axis=2).reshape(_ROWS, 168)


def _conv1(h0, w1a_ref, w1b_ref):
    parts = []
    for C in range(7):
        g = h0[:, C * 512:(C + 1) * 512]
        parts.append(jnp.dot(g[:, 0:256], w1a_ref[...],
                             preferred_element_type=jnp.float32))
        parts.append(jnp.dot(g[:, 256:512], w1b_ref[...],
                             preferred_element_type=jnp.float32))
    return jnp.concatenate(parts, axis=1)


def _conv2(h1, w2_ref):
    parts = []
    for C in range(7):
        g = h1[:, C * 256:(C + 1) * 256]
        parts.append(jnp.dot(g, w2_ref[...],
                             preferred_element_type=jnp.float32))
    return jnp.concatenate(parts, axis=1)


def _k_stats0(xs_ref, w_ref, m_ref, s_ref, q_ref):
    h = jnp.dot(_build_x(xs_ref).astype(_BF), w_ref[...],
                preferred_element_type=jnp.float32)
    hm = h * m_ref[...]
    s_ref[0, 0, :] = jnp.sum(hm, axis=0)
    q_ref[0, 0, :] = jnp.sum(hm * h, axis=0)


def _bn_vecs(s_ref, q_ref, f_ref, g_ref, n, ga_ref, be_ref):
    s = jnp.sum(s_ref[...], axis=(0, 1)).reshape(1, -1)
    q = jnp.sum(q_ref[...], axis=(0, 1)).reshape(1, -1)
    sc = jnp.dot(s, f_ref[...], preferred_element_type=jnp.float32) / n
    qc = jnp.dot(q, f_ref[...], preferred_element_type=jnp.float32) / n
    var = qc - sc * sc
    scale = ga_ref[...] * jax.lax.rsqrt(var + _EPS)
    shift = be_ref[...] - sc * scale
    sct = jnp.dot(scale, g_ref[...], preferred_element_type=jnp.float32)
    sht = jnp.dot(shift, g_ref[...], preferred_element_type=jnp.float32)
    return sct, sht


def _k_stage1(xs_ref, w0_ref, s0_ref, q0_ref, f0_ref, g0_ref, ga_ref, be_ref,
              w1a_ref, w1b_ref, m_ref, h1_ref, s_ref, q_ref):
    sct, sht = _bn_vecs(s0_ref, q0_ref, f0_ref, g0_ref, _N0, ga_ref, be_ref)
    h0raw = jnp.dot(_build_x(xs_ref).astype(_BF), w0_ref[...],
                    preferred_element_type=jnp.float32)
    h0 = jnp.maximum(h0raw * sct + sht, 0.0).astype(_BF)
    h1 = _conv1(h0, w1a_ref, w1b_ref)
    h1b = h1.astype(_BF)
    h1_ref[...] = h1b
    h1f = h1b.astype(jnp.float32)
    hm = h1f * m_ref[...]
    s_ref[0, 0, :] = jnp.sum(hm, axis=0)
    q_ref[0, 0, :] = jnp.sum(hm * h1f, axis=0)


def _k_stage2(h1_ref, s1_ref, q1_ref, f1_ref, g1_ref, ga_ref, be_ref,
              w2_ref, m_ref, h2_ref, s_ref, q_ref):
    sct, sht = _bn_vecs(s1_ref, q1_ref, f1_ref, g1_ref, _N1, ga_ref, be_ref)
    h1 = jnp.maximum(h1_ref[...].astype(jnp.float32) * sct + sht,
                     0.0).astype(_BF)
    h2 = _conv2(h1, w2_ref)
    h2b = h2.astype(_BF)
    h2_ref[...] = h2b
    h2f = h2b.astype(jnp.float32)
    hm = h2f * m_ref[...]
    s_ref[0, 0, :] = jnp.sum(hm, axis=0)
    q_ref[0, 0, :] = jnp.sum(hm * h2f, axis=0)


def _k_stage3(h2_ref, s2_ref, q2_ref, f2_ref, g2_ref, ga_ref, be_ref,
              ap_ref, wfc_ref, bfc_ref, o_ref):
    sct, sht = _bn_vecs(s2_ref, q2_ref, f2_ref, g2_ref, _N2, ga_ref, be_ref)
    h2 = jnp.maximum(h2_ref[...].astype(jnp.float32) * sct + sht, 0.0)
    acc = h2[:, 0:128]
    for C in range(1, 7):
        acc = acc + h2[:, C * 128:(C + 1) * 128]
    pooled = jnp.dot(ap_ref[...], acc, preferred_element_type=jnp.float32)
    o_ref[...] = jnp.dot(pooled, wfc_ref[...],
                         preferred_element_type=jnp.float32) + bfc_ref[0]


def kernel(x, W0, b0, g0, be0, W1, b1, g1, be1, W2, b2, g2, be2, Wfc, bfc):
    B = x.shape[0]
    nT = B // _T
    f32 = jnp.float32

    xs = jnp.pad(x.reshape(B, 7, 112), ((0, 0), (1, 8), (0, 0)))

    w0r = jnp.transpose(W0[:, 0], (1, 2, 0)).reshape(9, 32)
    W0g = jnp.einsum('kpn,nc->kpc', jnp.asarray(_S0),
                     w0r).reshape(168, 3584).astype(_BF)
    W1g = jnp.einsum('pqde,ocde->pcqo', jnp.asarray(_S1),
                     W1).reshape(512, 256).astype(_BF)
    W1a = W1g[0:256, 0:128]
    W1b = W1g[256:512, 128:256]
    W2g = jnp.transpose(W2, (2, 3, 1, 0)).reshape(256, 128).astype(_BF)
    wfcT = jnp.transpose(Wfc)
    apool = jnp.asarray(_APOOL)
    mask = jnp.asarray(_BMASK)
    F0, G0 = jnp.asarray(_F0), jnp.asarray(_G0)
    F1, G1 = jnp.asarray(_F1), jnp.asarray(_G1)
    F2, G2 = jnp.asarray(_F2), jnp.asarray(_G2)

    cparams = pltpu.CompilerParams(dimension_semantics=("parallel",))

    s0, q0 = pl.pallas_call(
        _k_stats0,
        grid=(nT,),
        in_specs=[
            pl.BlockSpec((_T, 16, 112), lambda i: (i, 0, 0)),
            pl.BlockSpec((168, 3584), lambda i: (0, 0)),
            pl.BlockSpec((_ROWS, 1), lambda i: (0, 0)),
        ],
        out_specs=[
            pl.BlockSpec((1, 1, 3584), lambda i: (i, 0, 0)),
            pl.BlockSpec((1, 1, 3584), lambda i: (i, 0, 0)),
        ],
        out_shape=[
            jax.ShapeDtypeStruct((nT, 1, 3584), f32),
            jax.ShapeDtypeStruct((nT, 1, 3584), f32),
        ],
        compiler_params=cparams,
    )(xs, W0g, mask)

    h1p, s1, q1 = pl.pallas_call(
        _k_stage1,
        grid=(nT,),
        in_specs=[
            pl.BlockSpec((_T, 16, 112), lambda i: (i, 0, 0)),
            pl.BlockSpec((168, 3584), lambda i: (0, 0)),
            pl.BlockSpec((nT, 1, 3584), lambda i: (0, 0, 0)),
            pl.BlockSpec((nT, 1, 3584), lambda i: (0, 0, 0)),
            pl.BlockSpec((3584, 32), lambda i: (0, 0)),
            pl.BlockSpec((32, 3584), lambda i: (0, 0)),
            pl.BlockSpec((1, 32), lambda i: (0, 0)),
            pl.BlockSpec((1, 32), lambda i: (0, 0)),
            pl.BlockSpec((256, 128), lambda i: (0, 0)),
            pl.BlockSpec((256, 128), lambda i: (0, 0)),
            pl.BlockSpec((_ROWS, 1), lambda i: (0, 0)),
        ],
        out_specs=[
            pl.BlockSpec((_ROWS, 1792), lambda i: (i, 0)),
            pl.BlockSpec((1, 1, 1792), lambda i: (i, 0, 0)),
            pl.BlockSpec((1, 1, 1792), lambda i: (i, 0, 0)),
        ],
        out_shape=[
            jax.ShapeDtypeStruct((B * 8, 1792), _BF),
            jax.ShapeDtypeStruct((nT, 1, 1792), f32),
            jax.ShapeDtypeStruct((nT, 1, 1792), f32),
        ],
        compiler_params=cparams,
    )(xs, W0g, s0, q0, F0, G0, g0.reshape(1, 32), be0.reshape(1, 32),
      W1a, W1b, mask)

    h2p, s2, q2 = pl.pallas_call(
        _k_stage2,
        grid=(nT,),
        in_specs=[
            pl.BlockSpec((_ROWS, 1792), lambda i: (i, 0)),
            pl.BlockSpec((nT, 1, 1792), lambda i: (0, 0, 0)),
            pl.BlockSpec((nT, 1, 1792), lambda i: (0, 0, 0)),
            pl.BlockSpec((1792, 64), lambda i: (0, 0)),
            pl.BlockSpec((64, 1792), lambda i: (0, 0)),
            pl.BlockSpec((1, 64), lambda i: (0, 0)),
            pl.BlockSpec((1, 64), lambda i: (0, 0)),
            pl.BlockSpec((256, 128), lambda i: (0, 0)),
            pl.BlockSpec((_ROWS, 1), lambda i: (0, 0)),
        ],
        out_specs=[
            pl.BlockSpec((_ROWS, 896), lambda i: (i, 0)),
            pl.BlockSpec((1, 1, 896), lambda i: (i, 0, 0)),
            pl.BlockSpec((1, 1, 896), lambda i: (i, 0, 0)),
        ],
        out_shape=[
            jax.ShapeDtypeStruct((B * 8, 896), _BF),
            jax.ShapeDtypeStruct((nT, 1, 896), f32),
            jax.ShapeDtypeStruct((nT, 1, 896), f32),
        ],
        compiler_params=cparams,
    )(h1p, s1, q1, F1, G1, g1.reshape(1, 64), be1.reshape(1, 64), W2g, mask)

    out = pl.pallas_call(
        _k_stage3,
        grid=(nT,),
        in_specs=[
            pl.BlockSpec((_ROWS, 896), lambda i: (i, 0)),
            pl.BlockSpec((nT, 1, 896), lambda i: (0, 0, 0)),
            pl.BlockSpec((nT, 1, 896), lambda i: (0, 0, 0)),
            pl.BlockSpec((896, 128), lambda i: (0, 0)),
            pl.BlockSpec((128, 896), lambda i: (0, 0)),
            pl.BlockSpec((1, 128), lambda i: (0, 0)),
            pl.BlockSpec((1, 128), lambda i: (0, 0)),
            pl.BlockSpec((_T, _ROWS), lambda i: (0, 0)),
            pl.BlockSpec((128, 10), lambda i: (0, 0)),
            pl.BlockSpec((1, 10), lambda i: (0, 0)),
        ],
        out_specs=pl.BlockSpec((_T, 10), lambda i: (i, 0)),
        out_shape=jax.ShapeDtypeStruct((B, 10), f32),
        compiler_params=cparams,
    )(h2p, s2, q2, F2, G2, g2.reshape(1, 128), be2.reshape(1, 128),
      apool, wfcT, bfc.reshape(1, 10))
    return out


# speedup vs baseline: 1.3871x; 1.0813x over previous
import numpy as np
import jax
import jax.numpy as jnp
from jax.experimental import pallas as pl
from jax.experimental.pallas import tpu as pltpu

_EPS = 1e-5
_BF = jnp.bfloat16
_N0 = 1024.0 * 28 * 28
_N1 = 1024.0 * 14 * 14
_N2 = 1024.0 * 7 * 7
_T = 256
_ROWS = _T * 8

_S0 = np.zeros((168, 7 * 16, 9), np.float32)
for C in range(7):
    for ue in range(4):
        for vf in range(4):
            for i in range(3):
                for j in range(3):
                    d = ue + i - 1
                    cx = 4 * C + vf + j - 1
                    if not (0 <= cx < 28):
                        continue
                    if d == -1:
                        k = cx
                    elif d <= 3:
                        k = 28 + 28 * d + cx
                    else:
                        k = 140 + cx
                    _S0[k, C * 16 + ue * 4 + vf, 3 * i + j] = 1.0
_S1 = np.zeros((16, 4, 2, 2), np.float32)
for e in range(2):
    for f in range(2):
        for dr in range(2):
            for dc in range(2):
                _S1[(2 * e + dr) * 4 + (2 * f + dc), e * 2 + f, dr, dc] = 1.0
_BMASK = np.tile(np.array([1.0] * 7 + [0.0], np.float32).reshape(8, 1),
                 (_T, 1))
_APOOL = np.kron(np.eye(_T, dtype=np.float32),
                 np.array([[1.0 / 49.0] * 7 + [0.0]], np.float32))


def _fold_mats(groups, ch):
    F = np.kron(np.ones((groups, 1), np.float32), np.eye(ch, dtype=np.float32))
    return F, F.T.copy()


_F0, _G0 = _fold_mats(112, 32)
_F1, _G1 = _fold_mats(28, 64)
_F2, _G2 = _fold_mats(7, 128)


def _build_x(xs_ref):
    xb = xs_ref[...]
    A = xb[:, 0:8, 84:112]
    Bv = xb[:, 1:9, :]
    Cv = xb[:, 2:10, 0:28]
    return jnp.concatenate([A, Bv, Cv], axis=2).reshape(_ROWS, 168)


def _conv1(h0, w1a_ref, w1b_ref):
    parts = []
    for C in range(7):
        g = h0[:, C * 512:(C + 1) * 512]
        parts.append(jnp.dot(g[:, 0:256], w1a_ref[...],
                             preferred_element_type=jnp.float32))
        parts.append(jnp.dot(g[:, 256:512], w1b_ref[...],
                             preferred_element_type=jnp.float32))
    return jnp.concatenate(parts, axis=1)


def _conv2(h1, w2_ref):
    parts = []
    for C in range(7):
        g = h1[:, C * 256:(C + 1) * 256]
        parts.append(jnp.dot(g, w2_ref[...],
                             preferred_element_type=jnp.float32))
    return jnp.concatenate(parts, axis=1)


def _k_stats0(xs_ref, w_ref, s_ref, q_ref):
    X = _build_x(xs_ref)
    G = jax.lax.dot_general(X, X, (((0,), (0,)), ((), ())),
                            preferred_element_type=jnp.float32)
    W = w_ref[...]
    GW = jnp.dot(G, W, preferred_element_type=jnp.float32)
    s_ref[0, 0, :] = jnp.dot(jnp.sum(X, axis=0, keepdims=True), W,
                             preferred_element_type=jnp.float32)[0]
    q_ref[0, 0, :] = jnp.sum(GW * W, axis=0)


def _bn_vecs(s_ref, q_ref, f_ref, g_ref, n, ga_ref, be_ref):
    s = jnp.sum(s_ref[...], axis=(0, 1)).reshape(1, -1)
    q = jnp.sum(q_ref[...], axis=(0, 1)).reshape(1, -1)
    sc = jnp.dot(s, f_ref[...], preferred_element_type=jnp.float32) / n
    qc = jnp.dot(q, f_ref[...], preferred_element_type=jnp.float32) / n
    var = qc - sc * sc
    scale = ga_ref[...] * jax.lax.rsqrt(var + _EPS)
    shift = be_ref[...] - sc * scale
    sct = jnp.dot(scale, g_ref[...], preferred_element_type=jnp.float32)
    sht = jnp.dot(shift, g_ref[...], preferred_element_type=jnp.float32)
    return sct, sht


def _k_stage1(xs_ref, w0_ref, s0_ref, q0_ref, f0_ref, g0_ref, ga_ref, be_ref,
              w1a_ref, w1b_ref, m_ref, h1_ref, s_ref, q_ref):
    sct, sht = _bn_vecs(s0_ref, q0_ref, f0_ref, g0_ref, _N0, ga_ref, be_ref)
    h0raw = jnp.dot(_build_x(xs_ref).astype(_BF), w0_ref[...],
                    preferred_element_type=jnp.float32)
    h0 = jnp.maximum(h0raw * sct + sht, 0.0).astype(_BF)
    h1 = _conv1(h0, w1a_ref, w1b_ref)
    h1b = h1.astype(_BF)
    h1_ref[...] = h1b
    h1f = h1b.astype(jnp.float32)
    hm = h1f * m_ref[...]
    s_ref[0, 0, :] = jnp.sum(hm, axis=0)
    q_ref[0, 0, :] = jnp.sum(hm * h1f, axis=0)


def _k_stage2(h1_ref, s1_ref, q1_ref, f1_ref, g1_ref, ga_ref, be_ref,
              w2_ref, m_ref, h2_ref, s_ref, q_ref):
    sct, sht = _bn_vecs(s1_ref, q1_ref, f1_ref, g1_ref, _N1, ga_ref, be_ref)
    h1 = jnp.maximum(h1_ref[...].astype(jnp.float32) * sct + sht,
                     0.0).astype(_BF)
    h2 = _conv2(h1, w2_ref)
    h2b = h2.astype(_BF)
    h2_ref[...] = h2b
    h2f = h2b.astype(jnp.float32)
    hm = h2f * m_ref[...]
    s_ref[0, 0, :] = jnp.sum(hm, axis=0)
    q_ref[0, 0, :] = jnp.sum(hm * h2f, axis=0)


def _k_stage3(h2_ref, s2_ref, q2_ref, f2_ref, g2_ref, ga_ref, be_ref,
              ap_ref, wfc_ref, bfc_ref, o_ref):
    sct, sht = _bn_vecs(s2_ref, q2_ref, f2_ref, g2_ref, _N2, ga_ref, be_ref)
    h2 = jnp.maximum(h2_ref[...].astype(jnp.float32) * sct + sht, 0.0)
    acc = h2[:, 0:128]
    for C in range(1, 7):
        acc = acc + h2[:, C * 128:(C + 1) * 128]
    pooled = jnp.dot(ap_ref[...], acc, preferred_element_type=jnp.float32)
    o_ref[...] = jnp.dot(pooled, wfc_ref[...],
                         preferred_element_type=jnp.float32) + bfc_ref[0]


def kernel(x, W0, b0, g0, be0, W1, b1, g1, be1, W2, b2, g2, be2, Wfc, bfc):
    B = x.shape[0]
    nT = B // _T
    f32 = jnp.float32

    xs = jnp.pad(x.reshape(B, 7, 112), ((0, 0), (1, 8), (0, 0)))

    w0r = jnp.transpose(W0[:, 0], (1, 2, 0)).reshape(9, 32)
    W0gf = jnp.einsum('kpn,nc->kpc', jnp.asarray(_S0),
                      w0r).reshape(168, 3584)
    W0g = W0gf.astype(_BF)
    W1g = jnp.einsum('pqde,ocde->pcqo', jnp.asarray(_S1),
                     W1).reshape(512, 256).astype(_BF)
    W1a = W1g[0:256, 0:128]
    W1b = W1g[256:512, 128:256]
    W2g = jnp.transpose(W2, (2, 3, 1, 0)).reshape(256, 128).astype(_BF)
    wfcT = jnp.transpose(Wfc)
    apool = jnp.asarray(_APOOL)
    mask = jnp.asarray(_BMASK)
    F0, G0 = jnp.asarray(_F0), jnp.asarray(_G0)
    F1, G1 = jnp.asarray(_F1), jnp.asarray(_G1)
    F2, G2 = jnp.asarray(_F2), jnp.asarray(_G2)

    cparams = pltpu.CompilerParams(dimension_semantics=("parallel",))

    s0, q0 = pl.pallas_call(
        _k_stats0,
        grid=(nT,),
        in_specs=[
            pl.BlockSpec((_T, 16, 112), lambda i: (i, 0, 0)),
            pl.BlockSpec((168, 3584), lambda i: (0, 0)),
        ],
        out_specs=[
            pl.BlockSpec((1, 1, 3584), lambda i: (i, 0, 0)),
            pl.BlockSpec((1, 1, 3584), lambda i: (i, 0, 0)),
        ],
        out_shape=[
            jax.ShapeDtypeStruct((nT, 1, 3584), f32),
            jax.ShapeDtypeStruct((nT, 1, 3584), f32),
        ],
        compiler_params=cparams,
    )(xs, W0gf)

    h1p, s1, q1 = pl.pallas_call(
        _k_stage1,
        grid=(nT,),
        in_specs=[
            pl.BlockSpec((_T, 16, 112), lambda i: (i, 0, 0)),
            pl.BlockSpec((168, 3584), lambda i: (0, 0)),
            pl.BlockSpec((nT, 1, 3584), lambda i: (0, 0, 0)),
            pl.BlockSpec((nT, 1, 3584), lambda i: (0, 0, 0)),
            pl.BlockSpec((3584, 32), lambda i: (0, 0)),
            pl.BlockSpec((32, 3584), lambda i: (0, 0)),
            pl.BlockSpec((1, 32), lambda i: (0, 0)),
            pl.BlockSpec((1, 32), lambda i: (0, 0)),
            pl.BlockSpec((256, 128), lambda i: (0, 0)),
            pl.BlockSpec((256, 128), lambda i: (0, 0)),
            pl.BlockSpec((_ROWS, 1), lambda i: (0, 0)),
        ],
        out_specs=[
            pl.BlockSpec((_ROWS, 1792), lambda i: (i, 0)),
            pl.BlockSpec((1, 1, 1792), lambda i: (i, 0, 0)),
            pl.BlockSpec((1, 1, 1792), lambda i: (i, 0, 0)),
        ],
        out_shape=[
            jax.ShapeDtypeStruct((B * 8, 1792), _BF),
            jax.ShapeDtypeStruct((nT, 1, 1792), f32),
            jax.ShapeDtypeStruct((nT, 1, 1792), f32),
        ],
        compiler_params=cparams,
    )(xs, W0g, s0, q0, F0, G0, g0.reshape(1, 32), be0.reshape(1, 32),
      W1a, W1b, mask)

    h2p, s2, q2 = pl.pallas_call(
        _k_stage2,
        grid=(nT,),
        in_specs=[
            pl.BlockSpec((_ROWS, 1792), lambda i: (i, 0)),
            pl.BlockSpec((nT, 1, 1792), lambda i: (0, 0, 0)),
            pl.BlockSpec((nT, 1, 1792), lambda i: (0, 0, 0)),
            pl.BlockSpec((1792, 64), lambda i: (0, 0)),
            pl.BlockSpec((64, 1792), lambda i: (0, 0)),
            pl.BlockSpec((1, 64), lambda i: (0, 0)),
            pl.BlockSpec((1, 64), lambda i: (0, 0)),
            pl.BlockSpec((256, 128), lambda i: (0, 0)),
            pl.BlockSpec((_ROWS, 1), lambda i: (0, 0)),
        ],
        out_specs=[
            pl.BlockSpec((_ROWS, 896), lambda i: (i, 0)),
            pl.BlockSpec((1, 1, 896), lambda i: (i, 0, 0)),
            pl.BlockSpec((1, 1, 896), lambda i: (i, 0, 0)),
        ],
        out_shape=[
            jax.ShapeDtypeStruct((B * 8, 896), _BF),
            jax.ShapeDtypeStruct((nT, 1, 896), f32),
            jax.ShapeDtypeStruct((nT, 1, 896), f32),
        ],
        compiler_params=cparams,
    )(h1p, s1, q1, F1, G1, g1.reshape(1, 64), be1.reshape(1, 64), W2g, mask)

    out = pl.pallas_call(
        _k_stage3,
        grid=(nT,),
        in_specs=[
            pl.BlockSpec((_ROWS, 896), lambda i: (i, 0)),
            pl.BlockSpec((nT, 1, 896), lambda i: (0, 0, 0)),
            pl.BlockSpec((nT, 1, 896), lambda i: (0, 0, 0)),
            pl.BlockSpec((896, 128), lambda i: (0, 0)),
            pl.BlockSpec((128, 896), lambda i: (0, 0)),
            pl.BlockSpec((1, 128), lambda i: (0, 0)),
            pl.BlockSpec((1, 128), lambda i: (0, 0)),
            pl.BlockSpec((_T, _ROWS), lambda i: (0, 0)),
            pl.BlockSpec((128, 10), lambda i: (0, 0)),
            pl.BlockSpec((1, 10), lambda i: (0, 0)),
        ],
        out_specs=pl.BlockSpec((_T, 10), lambda i: (i, 0)),
        out_shape=jax.ShapeDtypeStruct((B, 10), f32),
        compiler_params=cparams,
    )(h2p, s2, q2, F2, G2, g2.reshape(1, 128), be2.reshape(1, 128),
      apool, wfcT, bfc.reshape(1, 10))
    return out
